# baseline (speedup 1.0000x reference)
import functools

import jax
import jax.numpy as jnp
from jax import lax
from jax.experimental import pallas as pl
from jax.experimental.pallas import tpu as pltpu
from jax.experimental.pallas import tpu_sc as plsc

B = 16384
EMB = 16
CHUNK = 128

OFF_W1 = 0
OFF_W2 = 2048
OFF_W3 = 4096
OFF_W4 = 4608
OFF_WO = 4864
OFF_B1 = 4896
OFF_B2 = 4960
OFF_B3 = 4992
OFF_B4 = 5008
OFF_BO = 5024
WFLAT = 5056


def _neumf_body(bpw, nc, user_hbm, item_hbm, ugt, igt, umt, imt, wflat_hbm,
                out_hbm, uidx, iidx, ug_t, ig_t, um_t, im_t, w_v, out_v, sem):
    wid = lax.axis_index("s") * nc + lax.axis_index("c")
    nchunk = bpw // CHUNK
    base = wid * bpw
    iota = lax.iota(jnp.int32, 16)
    zeros = jnp.zeros((16,), jnp.float32)

    pltpu.sync_copy(user_hbm.at[pl.ds(wid * nchunk, nchunk)], uidx)
    pltpu.sync_copy(item_hbm.at[pl.ds(wid * nchunk, nchunk)], iidx)

    copies = []
    for j in range(nchunk):
        dst = pl.ds(j * CHUNK, CHUNK)
        for c in range(EMB):
            copies.append(pltpu.async_copy(
                ugt.at[c].at[uidx.at[j]], ug_t.at[c].at[dst], sem))
            copies.append(pltpu.async_copy(
                igt.at[c].at[iidx.at[j]], ig_t.at[c].at[dst], sem))
            copies.append(pltpu.async_copy(
                umt.at[c].at[uidx.at[j]], um_t.at[c].at[dst], sem))
            copies.append(pltpu.async_copy(
                imt.at[c].at[iidx.at[j]], im_t.at[c].at[dst], sem))

    pltpu.sync_copy(wflat_hbm, w_v)

    wm = w_v[pl.ds(OFF_WO + 16, 16)]
    wg = w_v[pl.ds(OFF_WO, 16)]

    def matvec_half(w_off, ncols, h, s):
        acc = zeros
        for j in range(ncols):
            col = plsc.load_gather(w_v, [w_off + (h * 16 + iota) * ncols + j])
            acc = acc + s[j // 16][j % 16] * col
        return acc

    s4 = matvec_half(OFF_W4, 16, 0, [wm])
    s3 = [matvec_half(OFF_W3, 16, h, [s4]) for h in range(2)]
    s2 = [matvec_half(OFF_W2, 32, q, s3) for q in range(4)]
    s1 = [matvec_half(OFF_W1, 64, h, s2) for h in range(2)]
    vu, vi = s1[0], s1[1]

    cv = zeros
    for q in range(4):
        cv = cv + w_v[pl.ds(OFF_B1 + 16 * q, 16)] * s2[q]
    for h in range(2):
        cv = cv + w_v[pl.ds(OFF_B2 + 16 * h, 16)] * s3[h]
    cv = cv + w_v[pl.ds(OFF_B3, 16)] * s4
    cv = cv + w_v[pl.ds(OFF_B4, 16)] * wm
    c = jnp.sum(cv) + w_v[pl.ds(OFF_BO, 16)][0]

    for cp in copies:
        cp.wait()

    def blk_step(b, carry):
        sl = pl.ds(b * 16, 16)
        acc = jnp.full((16,), c, jnp.float32)
        for k in range(EMB):
            u = ug_t[k, sl]
            g = ig_t[k, sl]
            mu = um_t[k, sl]
            mi = im_t[k, sl]
            acc = acc + (u * g) * wg[k] + mu * vu[k] + mi * vi[k]
        out_v[sl] = 1.0 / (1.0 + jnp.exp(-acc))
        return carry

    lax.fori_loop(0, bpw // 16, blk_step, 0)

    pltpu.sync_copy(out_v, out_hbm.at[pl.ds(base, bpw)])


def kernel(user, item, user_GMF, item_GMF, user_MLP, item_MLP,
           W1, b1, W2, b2, W3, b3, W4, b4, Wo, bo):
    mesh = plsc.VectorSubcoreMesh(core_axis_name="c", subcore_axis_name="s")
    nc, ns = mesh.num_cores, mesh.num_subcores
    nw = nc * ns
    assert B % (nw * CHUNK) == 0
    bpw = B // nw

    pad = jnp.zeros((WFLAT - 5025,), jnp.float32)
    wflat = jnp.concatenate([
        W1.reshape(-1), W2.reshape(-1), W3.reshape(-1), W4.reshape(-1),
        Wo.reshape(-1), b1, b2, b3, b4, bo, pad])
    user2 = user.astype(jnp.int32).reshape(B // CHUNK, CHUNK)
    item2 = item.astype(jnp.int32).reshape(B // CHUNK, CHUNK)

    nchunk = bpw // CHUNK
    f = pl.kernel(
        functools.partial(_neumf_body, bpw, nc),
        out_type=jax.ShapeDtypeStruct((B,), jnp.float32),
        mesh=mesh,
        compiler_params=pltpu.CompilerParams(
            needs_layout_passes=False, use_tc_tiling_on_sc=False),
        scratch_types=[
            pltpu.VMEM((nchunk, CHUNK), jnp.int32),
            pltpu.VMEM((nchunk, CHUNK), jnp.int32),
            pltpu.VMEM((EMB, bpw), jnp.float32),
            pltpu.VMEM((EMB, bpw), jnp.float32),
            pltpu.VMEM((EMB, bpw), jnp.float32),
            pltpu.VMEM((EMB, bpw), jnp.float32),
            pltpu.VMEM((WFLAT,), jnp.float32),
            pltpu.VMEM((bpw,), jnp.float32),
            pltpu.SemaphoreType.DMA,
        ],
    )
    out = f(user2, item2, user_GMF.T, item_GMF.T, user_MLP.T, item_MLP.T,
            wflat)
    return out.reshape(B, 1)


# speedup vs baseline: 4.8840x; 4.8840x over previous
import functools

import jax
import jax.numpy as jnp
from jax import lax
from jax.experimental import pallas as pl
from jax.experimental.pallas import tpu as pltpu
from jax.experimental.pallas import tpu_sc as plsc

B = 16384
EMB = 16
CHUNK = 128

V = 1000001
TBLK = 8192
NSTEP = (V + TBLK - 1) // TBLK
PROWS = NSTEP * TBLK // 8

OFF_W1 = 0
OFF_W2 = 2048
OFF_W3 = 4096
OFF_W4 = 4608
OFF_WO = 4864
OFF_B1 = 4896
OFF_B2 = 4960
OFF_B3 = 4992
OFF_B4 = 5008
OFF_BO = 5024
WFLAT = 5056


def _detile_body(x1, x2, x3, x4, o1, o2, o3, o4):
    for x, o in ((x1, o1), (x2, o2), (x3, o3), (x4, o4)):
        blk = x[...]
        parts = [blk[:, p * 1024:(p + 1) * 1024].T for p in range(8)]
        o[...] = jnp.concatenate(parts, axis=1)


_detile = pl.pallas_call(
    _detile_body,
    grid=(NSTEP,),
    in_specs=[pl.BlockSpec((EMB, TBLK), lambda j: (0, j))] * 4,
    out_specs=[pl.BlockSpec((TBLK // 8, 128), lambda j: (j, 0))] * 4,
    out_shape=[jax.ShapeDtypeStruct((PROWS, 128), jnp.float32)] * 4,
)


def _neumf_body(bpw, nc, user_hbm, item_hbm, ugt, igt, umt, imt, wflat_hbm,
                out_hbm, uidx, iidx, uq, iq, bufs, w_v, out_v, sem):
    wid = lax.axis_index("s") * nc + lax.axis_index("c")
    nchunk = bpw // CHUNK
    base = wid * bpw
    iota = lax.iota(jnp.int32, 16)
    zeros = jnp.zeros((16,), jnp.float32)

    pltpu.sync_copy(user_hbm.at[pl.ds(wid * nchunk, nchunk)], uidx)
    pltpu.sync_copy(item_hbm.at[pl.ds(wid * nchunk, nchunk)], iidx)
    def packed_row(v):
        return lax.shift_left(lax.shift_right_logical(v, 13), 10) | (v & 1023)

    for j in range(nchunk):
        for h in range(CHUNK // 16):
            sl = pl.ds(h * 16, 16)
            uq[j, sl] = packed_row(uidx[j, sl])
            iq[j, sl] = packed_row(iidx[j, sl])

    ubuf, ibuf, mubuf, mibuf = bufs

    def fire(j):
        return [
            pltpu.async_copy(ugt.at[uq.at[j]], ubuf, sem),
            pltpu.async_copy(igt.at[iq.at[j]], ibuf, sem),
            pltpu.async_copy(umt.at[uq.at[j]], mubuf, sem),
            pltpu.async_copy(imt.at[iq.at[j]], mibuf, sem),
        ]

    inflight = fire(0)

    pltpu.sync_copy(wflat_hbm, w_v)

    wm = w_v[pl.ds(OFF_WO + 16, 16)]
    wg = w_v[pl.ds(OFF_WO, 16)]

    def matvec_half(w_off, ncols, h, s):
        acc = zeros
        for j in range(ncols):
            col = plsc.load_gather(w_v, [w_off + (h * 16 + iota) * ncols + j])
            acc = acc + s[j // 16][j % 16] * col
        return acc

    s4 = matvec_half(OFF_W4, 16, 0, [wm])
    s3 = [matvec_half(OFF_W3, 16, h, [s4]) for h in range(2)]
    s2 = [matvec_half(OFF_W2, 32, q, s3) for q in range(4)]
    s1 = [matvec_half(OFF_W1, 64, h, s2) for h in range(2)]
    vu, vi = s1[0], s1[1]

    cv = zeros
    for q in range(4):
        cv = cv + w_v[pl.ds(OFF_B1 + 16 * q, 16)] * s2[q]
    for h in range(2):
        cv = cv + w_v[pl.ds(OFF_B2 + 16 * h, 16)] * s3[h]
    cv = cv + w_v[pl.ds(OFF_B3, 16)] * s4
    cv = cv + w_v[pl.ds(OFF_B4, 16)] * wm
    c = jnp.sum(cv) + w_v[pl.ds(OFF_BO, 16)][0]

    for j in range(nchunk):
        for cp in inflight:
            cp.wait()

        def blk_step(b, carry, j=j):
            sl = pl.ds(b * 16, 16)
            rows = b * 16 + iota
            ru = (lax.shift_right_logical(uidx[j, sl], 10) & 7) * 16
            ri = (lax.shift_right_logical(iidx[j, sl], 10) & 7) * 16
            acc = jnp.full((16,), c, jnp.float32)
            for k in range(EMB):
                u = plsc.load_gather(ubuf, [rows, ru + k])
                g = plsc.load_gather(ibuf, [rows, ri + k])
                mu = plsc.load_gather(mubuf, [rows, ru + k])
                mi = plsc.load_gather(mibuf, [rows, ri + k])
                acc = acc + (u * g) * wg[k] + mu * vu[k] + mi * vi[k]
            out_v[pl.ds(j * CHUNK + b * 16, 16)] = 1.0 / (1.0 + jnp.exp(-acc))
            return carry

        lax.fori_loop(0, CHUNK // 16, blk_step, 0)
        if j + 1 < nchunk:
            inflight = fire(j + 1)

    pltpu.sync_copy(out_v, out_hbm.at[pl.ds(base, bpw)])


def kernel(user, item, user_GMF, item_GMF, user_MLP, item_MLP,
           W1, b1, W2, b2, W3, b3, W4, b4, Wo, bo):
    mesh = plsc.VectorSubcoreMesh(core_axis_name="c", subcore_axis_name="s")
    nc, ns = mesh.num_cores, mesh.num_subcores
    nw = nc * ns
    assert B % (nw * CHUNK) == 0
    bpw = B // nw

    pad = jnp.zeros((WFLAT - 5025,), jnp.float32)
    wflat = jnp.concatenate([
        W1.reshape(-1), W2.reshape(-1), W3.reshape(-1), W4.reshape(-1),
        Wo.reshape(-1), b1, b2, b3, b4, bo, pad])
    user2 = user.astype(jnp.int32).reshape(B // CHUNK, CHUNK)
    item2 = item.astype(jnp.int32).reshape(B // CHUNK, CHUNK)

    ugp, igp, ump, imp = _detile(
        user_GMF.T, item_GMF.T, user_MLP.T, item_MLP.T)

    nchunk = bpw // CHUNK
    f = pl.kernel(
        functools.partial(_neumf_body, bpw, nc),
        out_type=jax.ShapeDtypeStruct((B,), jnp.float32),
        mesh=mesh,
        compiler_params=pltpu.CompilerParams(
            needs_layout_passes=False, use_tc_tiling_on_sc=False),
        scratch_types=[
            pltpu.VMEM((nchunk, CHUNK), jnp.int32),
            pltpu.VMEM((nchunk, CHUNK), jnp.int32),
            pltpu.VMEM((nchunk, CHUNK), jnp.int32),
            pltpu.VMEM((nchunk, CHUNK), jnp.int32),
            [pltpu.VMEM((CHUNK, 128), jnp.float32)] * 4,
            pltpu.VMEM((WFLAT,), jnp.float32),
            pltpu.VMEM((bpw,), jnp.float32),
            pltpu.SemaphoreType.DMA,
        ],
    )
    out = f(user2, item2, ugp, igp, ump, imp, wflat)
    return out.reshape(B, 1)


# speedup vs baseline: 21.5020x; 4.4025x over previous
import functools

import jax
import jax.numpy as jnp
from jax import lax
from jax.experimental import pallas as pl
from jax.experimental.pallas import tpu as pltpu
from jax.experimental.pallas import tpu_sc as plsc

B = 16384
EMB = 16
CHUNK = 128

V = 1000001
TBLK = 8192
NSTEP = (V + TBLK - 1) // TBLK
PROWS = NSTEP * TBLK // 8

OFF_W1 = 0
OFF_W2 = 2048
OFF_W3 = 4096
OFF_W4 = 4608
OFF_WO = 4864
OFF_B1 = 4896
OFF_B2 = 4960
OFF_B3 = 4992
OFF_B4 = 5008
OFF_BO = 5024
WFLAT = 5056


def _detile_body(x1, x2, x3, x4, o1, o2, o3, o4):
    for x, o in ((x1, o1), (x2, o2), (x3, o3), (x4, o4)):
        blk = x[...]
        z = jnp.concatenate(
            [blk[:, p * 1024:(p + 1) * 1024] for p in range(8)], axis=0)
        o[...] = z.T


_detile = pl.pallas_call(
    _detile_body,
    grid=(NSTEP,),
    in_specs=[pl.BlockSpec((EMB, TBLK), lambda j: (0, j))] * 4,
    out_specs=[pl.BlockSpec((TBLK // 8, 128), lambda j: (j, 0))] * 4,
    out_shape=[jax.ShapeDtypeStruct((PROWS, 128), jnp.float32)] * 4,
)


def _neumf_body(bpw, nc, user_hbm, item_hbm, ugt, igt, umt, imt, wflat_hbm,
                out_hbm, uidx, iidx, uq, iq, bufs, w_v, out_v, sem):
    wid = lax.axis_index("s") * nc + lax.axis_index("c")
    nchunk = bpw // CHUNK
    base = wid * bpw
    iota = lax.iota(jnp.int32, 16)
    zeros = jnp.zeros((16,), jnp.float32)

    pltpu.sync_copy(user_hbm.at[pl.ds(wid * nchunk, nchunk)], uidx)
    pltpu.sync_copy(item_hbm.at[pl.ds(wid * nchunk, nchunk)], iidx)
    def packed_row(v):
        return lax.shift_left(lax.shift_right_logical(v, 13), 10) | (v & 1023)

    for j in range(nchunk):
        for h in range(CHUNK // 16):
            sl = pl.ds(h * 16, 16)
            uq[j, sl] = packed_row(uidx[j, sl])
            iq[j, sl] = packed_row(iidx[j, sl])

    ubuf, ibuf, mubuf, mibuf = bufs

    def fire(j):
        return [
            pltpu.async_copy(ugt.at[uq.at[j]], ubuf, sem),
            pltpu.async_copy(igt.at[iq.at[j]], ibuf, sem),
            pltpu.async_copy(umt.at[uq.at[j]], mubuf, sem),
            pltpu.async_copy(imt.at[iq.at[j]], mibuf, sem),
        ]

    inflight = fire(0)

    pltpu.sync_copy(wflat_hbm, w_v)

    wm = w_v[pl.ds(OFF_WO + 16, 16)]
    wg = w_v[pl.ds(OFF_WO, 16)]

    def matvec_half(w_off, ncols, h, s):
        acc = zeros
        for j in range(ncols):
            col = plsc.load_gather(w_v, [w_off + (h * 16 + iota) * ncols + j])
            acc = acc + s[j // 16][j % 16] * col
        return acc

    s4 = matvec_half(OFF_W4, 16, 0, [wm])
    s3 = [matvec_half(OFF_W3, 16, h, [s4]) for h in range(2)]
    s2 = [matvec_half(OFF_W2, 32, q, s3) for q in range(4)]
    s1 = [matvec_half(OFF_W1, 64, h, s2) for h in range(2)]
    vu, vi = s1[0], s1[1]

    cv = zeros
    for q in range(4):
        cv = cv + w_v[pl.ds(OFF_B1 + 16 * q, 16)] * s2[q]
    for h in range(2):
        cv = cv + w_v[pl.ds(OFF_B2 + 16 * h, 16)] * s3[h]
    cv = cv + w_v[pl.ds(OFF_B3, 16)] * s4
    cv = cv + w_v[pl.ds(OFF_B4, 16)] * wm
    c = jnp.sum(cv) + w_v[pl.ds(OFF_BO, 16)][0]

    for j in range(nchunk):
        for cp in inflight:
            cp.wait()

        def blk_step(b, carry, j=j):
            sl = pl.ds(b * 16, 16)
            rows = b * 16 + iota
            ru = (lax.shift_right_logical(uidx[j, sl], 10) & 7) * 16
            ri = (lax.shift_right_logical(iidx[j, sl], 10) & 7) * 16
            acc = jnp.full((16,), c, jnp.float32)
            for k in range(EMB):
                u = plsc.load_gather(ubuf, [rows, ru + k])
                g = plsc.load_gather(ibuf, [rows, ri + k])
                mu = plsc.load_gather(mubuf, [rows, ru + k])
                mi = plsc.load_gather(mibuf, [rows, ri + k])
                acc = acc + (u * g) * wg[k] + mu * vu[k] + mi * vi[k]
            out_v[pl.ds(j * CHUNK + b * 16, 16)] = 1.0 / (1.0 + jnp.exp(-acc))
            return carry

        lax.fori_loop(0, CHUNK // 16, blk_step, 0)
        if j + 1 < nchunk:
            inflight = fire(j + 1)

    pltpu.sync_copy(out_v, out_hbm.at[pl.ds(base, bpw)])


def kernel(user, item, user_GMF, item_GMF, user_MLP, item_MLP,
           W1, b1, W2, b2, W3, b3, W4, b4, Wo, bo):
    mesh = plsc.VectorSubcoreMesh(core_axis_name="c", subcore_axis_name="s")
    nc, ns = mesh.num_cores, mesh.num_subcores
    nw = nc * ns
    assert B % (nw * CHUNK) == 0
    bpw = B // nw

    pad = jnp.zeros((WFLAT - 5025,), jnp.float32)
    wflat = jnp.concatenate([
        W1.reshape(-1), W2.reshape(-1), W3.reshape(-1), W4.reshape(-1),
        Wo.reshape(-1), b1, b2, b3, b4, bo, pad])
    user2 = user.astype(jnp.int32).reshape(B // CHUNK, CHUNK)
    item2 = item.astype(jnp.int32).reshape(B // CHUNK, CHUNK)

    ugp, igp, ump, imp = _detile(
        user_GMF.T, item_GMF.T, user_MLP.T, item_MLP.T)

    nchunk = bpw // CHUNK
    f = pl.kernel(
        functools.partial(_neumf_body, bpw, nc),
        out_type=jax.ShapeDtypeStruct((B,), jnp.float32),
        mesh=mesh,
        compiler_params=pltpu.CompilerParams(
            needs_layout_passes=False, use_tc_tiling_on_sc=False),
        scratch_types=[
            pltpu.VMEM((nchunk, CHUNK), jnp.int32),
            pltpu.VMEM((nchunk, CHUNK), jnp.int32),
            pltpu.VMEM((nchunk, CHUNK), jnp.int32),
            pltpu.VMEM((nchunk, CHUNK), jnp.int32),
            [pltpu.VMEM((CHUNK, 128), jnp.float32)] * 4,
            pltpu.VMEM((WFLAT,), jnp.float32),
            pltpu.VMEM((bpw,), jnp.float32),
            pltpu.SemaphoreType.DMA,
        ],
    )
    out = f(user2, item2, ugp, igp, ump, imp, wflat)
    return out.reshape(B, 1)


# speedup vs baseline: 24.4211x; 1.1358x over previous
import functools

import jax
import jax.numpy as jnp
from jax import lax
from jax.experimental import pallas as pl
from jax.experimental.pallas import tpu as pltpu
from jax.experimental.pallas import tpu_sc as plsc

B = 16384
EMB = 16
CHUNK = 128

V = 1000001
TBLK = 16384
GRP = TBLK // 8
LOG_TBLK = 14
LOG_GRP = 11
NSTEP = (V + TBLK - 1) // TBLK
PROWS = NSTEP * TBLK // 8

OFF_W1 = 0
OFF_W2 = 2048
OFF_W3 = 4096
OFF_W4 = 4608
OFF_WO = 4864
OFF_B1 = 4896
OFF_B2 = 4960
OFF_B3 = 4992
OFF_B4 = 5008
OFF_BO = 5024
WFLAT = 5056


def _detile_body(x1, x2, x3, x4, o1, o2, o3, o4):
    for x, o in ((x1, o1), (x2, o2), (x3, o3), (x4, o4)):
        blk = x[...]
        z = jnp.concatenate(
            [blk[:, p * GRP:(p + 1) * GRP] for p in range(8)], axis=0)
        o[...] = z.T


_detile = pl.pallas_call(
    _detile_body,
    grid=(NSTEP,),
    in_specs=[pl.BlockSpec((EMB, TBLK), lambda j: (0, j))] * 4,
    out_specs=[pl.BlockSpec((GRP, 128), lambda j: (j, 0))] * 4,
    out_shape=[jax.ShapeDtypeStruct((PROWS, 128), jnp.float32)] * 4,
)


def _neumf_body(bpw, nc, user_hbm, item_hbm, ugt, igt, umt, imt, wflat_hbm,
                out_hbm, uidx, iidx, uq, iq, bufs, w_v, out_v, sem):
    wid = lax.axis_index("s") * nc + lax.axis_index("c")
    nchunk = bpw // CHUNK
    base = wid * bpw
    iota = lax.iota(jnp.int32, 16)
    zeros = jnp.zeros((16,), jnp.float32)

    pltpu.sync_copy(user_hbm.at[pl.ds(wid * nchunk, nchunk)], uidx)
    pltpu.sync_copy(item_hbm.at[pl.ds(wid * nchunk, nchunk)], iidx)
    def packed_row(v):
        return (lax.shift_left(lax.shift_right_logical(v, LOG_TBLK), LOG_GRP)
                | (v & (GRP - 1)))

    for j in range(nchunk):
        for h in range(CHUNK // 16):
            sl = pl.ds(h * 16, 16)
            uq[j, sl] = packed_row(uidx[j, sl])
            iq[j, sl] = packed_row(iidx[j, sl])

    ubuf, ibuf, mubuf, mibuf = bufs

    def fire(j):
        return [
            pltpu.async_copy(ugt.at[uq.at[j]], ubuf, sem),
            pltpu.async_copy(igt.at[iq.at[j]], ibuf, sem),
            pltpu.async_copy(umt.at[uq.at[j]], mubuf, sem),
            pltpu.async_copy(imt.at[iq.at[j]], mibuf, sem),
        ]

    inflight = fire(0)

    pltpu.sync_copy(wflat_hbm, w_v)

    wm = w_v[pl.ds(OFF_WO + 16, 16)]
    wg = w_v[pl.ds(OFF_WO, 16)]

    def matvec_half(w_off, ncols, h, s):
        acc = zeros
        for j in range(ncols):
            col = plsc.load_gather(w_v, [w_off + (h * 16 + iota) * ncols + j])
            acc = acc + s[j // 16][j % 16] * col
        return acc

    s4 = matvec_half(OFF_W4, 16, 0, [wm])
    s3 = [matvec_half(OFF_W3, 16, h, [s4]) for h in range(2)]
    s2 = [matvec_half(OFF_W2, 32, q, s3) for q in range(4)]
    s1 = [matvec_half(OFF_W1, 64, h, s2) for h in range(2)]
    vu, vi = s1[0], s1[1]

    cv = zeros
    for q in range(4):
        cv = cv + w_v[pl.ds(OFF_B1 + 16 * q, 16)] * s2[q]
    for h in range(2):
        cv = cv + w_v[pl.ds(OFF_B2 + 16 * h, 16)] * s3[h]
    cv = cv + w_v[pl.ds(OFF_B3, 16)] * s4
    cv = cv + w_v[pl.ds(OFF_B4, 16)] * wm
    c = jnp.sum(cv) + w_v[pl.ds(OFF_BO, 16)][0]

    for j in range(nchunk):
        for cp in inflight:
            cp.wait()

        def blk_step(b, carry, j=j):
            sl = pl.ds(b * 16, 16)
            rows = b * 16 + iota
            ru = (lax.shift_right_logical(uidx[j, sl], LOG_GRP) & 7) * 16
            ri = (lax.shift_right_logical(iidx[j, sl], LOG_GRP) & 7) * 16
            acc = jnp.full((16,), c, jnp.float32)
            for k in range(EMB):
                u = plsc.load_gather(ubuf, [rows, ru + k])
                g = plsc.load_gather(ibuf, [rows, ri + k])
                mu = plsc.load_gather(mubuf, [rows, ru + k])
                mi = plsc.load_gather(mibuf, [rows, ri + k])
                acc = acc + (u * g) * wg[k] + mu * vu[k] + mi * vi[k]
            out_v[pl.ds(j * CHUNK + b * 16, 16)] = 1.0 / (1.0 + jnp.exp(-acc))
            return carry

        lax.fori_loop(0, CHUNK // 16, blk_step, 0)
        if j + 1 < nchunk:
            inflight = fire(j + 1)

    pltpu.sync_copy(out_v, out_hbm.at[pl.ds(base, bpw)])


def kernel(user, item, user_GMF, item_GMF, user_MLP, item_MLP,
           W1, b1, W2, b2, W3, b3, W4, b4, Wo, bo):
    mesh = plsc.VectorSubcoreMesh(core_axis_name="c", subcore_axis_name="s")
    nc, ns = mesh.num_cores, mesh.num_subcores
    nw = nc * ns
    assert B % (nw * CHUNK) == 0
    bpw = B // nw

    pad = jnp.zeros((WFLAT - 5025,), jnp.float32)
    wflat = jnp.concatenate([
        W1.reshape(-1), W2.reshape(-1), W3.reshape(-1), W4.reshape(-1),
        Wo.reshape(-1), b1, b2, b3, b4, bo, pad])
    user2 = user.astype(jnp.int32).reshape(B // CHUNK, CHUNK)
    item2 = item.astype(jnp.int32).reshape(B // CHUNK, CHUNK)

    ugp, igp, ump, imp = _detile(
        user_GMF.T, item_GMF.T, user_MLP.T, item_MLP.T)

    nchunk = bpw // CHUNK
    f = pl.kernel(
        functools.partial(_neumf_body, bpw, nc),
        out_type=jax.ShapeDtypeStruct((B,), jnp.float32),
        mesh=mesh,
        compiler_params=pltpu.CompilerParams(
            needs_layout_passes=False, use_tc_tiling_on_sc=False),
        scratch_types=[
            pltpu.VMEM((nchunk, CHUNK), jnp.int32),
            pltpu.VMEM((nchunk, CHUNK), jnp.int32),
            pltpu.VMEM((nchunk, CHUNK), jnp.int32),
            pltpu.VMEM((nchunk, CHUNK), jnp.int32),
            [pltpu.VMEM((CHUNK, 128), jnp.float32)] * 4,
            pltpu.VMEM((WFLAT,), jnp.float32),
            pltpu.VMEM((bpw,), jnp.float32),
            pltpu.SemaphoreType.DMA,
        ],
    )
    out = f(user2, item2, ugp, igp, ump, imp, wflat)
    return out.reshape(B, 1)


# speedup vs baseline: 30.0901x; 1.2321x over previous
import functools

import jax
import jax.numpy as jnp
from jax import lax
from jax.experimental import pallas as pl
from jax.experimental.pallas import tpu as pltpu
from jax.experimental.pallas import tpu_sc as plsc

B = 16384
EMB = 16
CHUNK = 128

V = 1000001
TBLK = 16384
GRP = TBLK // 16
LOG_TBLK = 14
LOG_GRP = 10
NSTEP = (V + TBLK - 1) // TBLK
PROWS = NSTEP * GRP

OFF_W1 = 0
OFF_W2 = 2048
OFF_W3 = 4096
OFF_W4 = 4608
OFF_WO = 4864
OFF_B1 = 4896
OFF_B2 = 4960
OFF_B3 = 4992
OFF_B4 = 5008
OFF_BO = 5024
WFLAT = 5056


def _detile_body(x1, x2, x3, x4, o1, o2, o3, o4):
    for x, o in ((x1, o1), (x2, o2), (x3, o3), (x4, o4)):
        blk = x[...]
        lo = jnp.concatenate(
            [blk[:, p * GRP:(p + 1) * GRP] for p in range(8)], axis=0)
        hi = jnp.concatenate(
            [blk[:, (p + 8) * GRP:(p + 9) * GRP] for p in range(8)], axis=0)
        au = lax.bitcast_convert_type(
            lo.astype(jnp.bfloat16), jnp.uint16).astype(jnp.uint32)
        bu = lax.bitcast_convert_type(
            hi.astype(jnp.bfloat16), jnp.uint16).astype(jnp.uint32)
        w = lax.bitcast_convert_type((bu << 16) | au, jnp.int32)
        o[...] = w.T


_detile = pl.pallas_call(
    _detile_body,
    grid=(NSTEP,),
    in_specs=[pl.BlockSpec((EMB, TBLK), lambda j: (0, j))] * 4,
    out_specs=[pl.BlockSpec((GRP, 128), lambda j: (j, 0))] * 4,
    out_shape=[jax.ShapeDtypeStruct((PROWS, 128), jnp.int32)] * 4,
)


def _neumf_body(bpw, nc, user_hbm, item_hbm, ugt, igt, umt, imt, wflat_hbm,
                out_hbm, uidx, iidx, uq, iq, bufs, w_v, out_v, sem):
    wid = lax.axis_index("s") * nc + lax.axis_index("c")
    nchunk = bpw // CHUNK
    base = wid * bpw
    iota = lax.iota(jnp.int32, 16)
    zeros = jnp.zeros((16,), jnp.float32)

    pltpu.sync_copy(user_hbm.at[pl.ds(wid * nchunk, nchunk)], uidx)
    pltpu.sync_copy(item_hbm.at[pl.ds(wid * nchunk, nchunk)], iidx)
    def packed_row(v):
        return (lax.shift_left(lax.shift_right_logical(v, LOG_TBLK), LOG_GRP)
                | (v & (GRP - 1)))

    for j in range(nchunk):
        for h in range(CHUNK // 16):
            sl = pl.ds(h * 16, 16)
            uq[j, sl] = packed_row(uidx[j, sl])
            iq[j, sl] = packed_row(iidx[j, sl])

    ubuf, ibuf, mubuf, mibuf = bufs

    def fire(j):
        return [
            pltpu.async_copy(ugt.at[uq.at[j]], ubuf, sem),
            pltpu.async_copy(igt.at[iq.at[j]], ibuf, sem),
            pltpu.async_copy(umt.at[uq.at[j]], mubuf, sem),
            pltpu.async_copy(imt.at[iq.at[j]], mibuf, sem),
        ]

    inflight = fire(0)

    pltpu.sync_copy(wflat_hbm, w_v)

    wm = w_v[pl.ds(OFF_WO + 16, 16)]
    wg = w_v[pl.ds(OFF_WO, 16)]

    def matvec_half(w_off, ncols, h, s):
        acc = zeros
        for j in range(ncols):
            col = plsc.load_gather(w_v, [w_off + (h * 16 + iota) * ncols + j])
            acc = acc + s[j // 16][j % 16] * col
        return acc

    s4 = matvec_half(OFF_W4, 16, 0, [wm])
    s3 = [matvec_half(OFF_W3, 16, h, [s4]) for h in range(2)]
    s2 = [matvec_half(OFF_W2, 32, q, s3) for q in range(4)]
    s1 = [matvec_half(OFF_W1, 64, h, s2) for h in range(2)]
    vu, vi = s1[0], s1[1]

    cv = zeros
    for q in range(4):
        cv = cv + w_v[pl.ds(OFF_B1 + 16 * q, 16)] * s2[q]
    for h in range(2):
        cv = cv + w_v[pl.ds(OFF_B2 + 16 * h, 16)] * s3[h]
    cv = cv + w_v[pl.ds(OFF_B3, 16)] * s4
    cv = cv + w_v[pl.ds(OFF_B4, 16)] * wm
    c = jnp.sum(cv) + w_v[pl.ds(OFF_BO, 16)][0]

    for j in range(nchunk):
        for cp in inflight:
            cp.wait()

        def blk_step(b, carry, j=j):
            sl = pl.ds(b * 16, 16)
            rows = b * 16 + iota
            up = lax.shift_right_logical(uidx[j, sl], LOG_GRP) & 15
            ip = lax.shift_right_logical(iidx[j, sl], LOG_GRP) & 15
            ru = (up & 7) * 16
            ri = (ip & 7) * 16
            usel = up > 7
            isel = ip > 7
            acc = jnp.full((16,), c, jnp.float32)
            hi_mask = jnp.full((16,), -65536, jnp.int32)

            def unpk(w2, sel):
                lo = plsc.bitcast(lax.shift_left(w2, 16), jnp.float32)
                hi = plsc.bitcast(w2 & hi_mask, jnp.float32)
                return jnp.where(sel, hi, lo)

            for k in range(EMB):
                u = unpk(plsc.load_gather(ubuf, [rows, ru + k]), usel)
                g = unpk(plsc.load_gather(ibuf, [rows, ri + k]), isel)
                mu = unpk(plsc.load_gather(mubuf, [rows, ru + k]), usel)
                mi = unpk(plsc.load_gather(mibuf, [rows, ri + k]), isel)
                acc = acc + (u * g) * wg[k] + mu * vu[k] + mi * vi[k]
            out_v[pl.ds(j * CHUNK + b * 16, 16)] = 1.0 / (1.0 + jnp.exp(-acc))
            return carry

        lax.fori_loop(0, CHUNK // 16, blk_step, 0)
        if j + 1 < nchunk:
            inflight = fire(j + 1)

    pltpu.sync_copy(out_v, out_hbm.at[pl.ds(base, bpw)])


def kernel(user, item, user_GMF, item_GMF, user_MLP, item_MLP,
           W1, b1, W2, b2, W3, b3, W4, b4, Wo, bo):
    mesh = plsc.VectorSubcoreMesh(core_axis_name="c", subcore_axis_name="s")
    nc, ns = mesh.num_cores, mesh.num_subcores
    nw = nc * ns
    assert B % (nw * CHUNK) == 0
    bpw = B // nw

    pad = jnp.zeros((WFLAT - 5025,), jnp.float32)
    wflat = jnp.concatenate([
        W1.reshape(-1), W2.reshape(-1), W3.reshape(-1), W4.reshape(-1),
        Wo.reshape(-1), b1, b2, b3, b4, bo, pad])
    user2 = user.astype(jnp.int32).reshape(B // CHUNK, CHUNK)
    item2 = item.astype(jnp.int32).reshape(B // CHUNK, CHUNK)

    ugp, igp, ump, imp = _detile(
        user_GMF.T, item_GMF.T, user_MLP.T, item_MLP.T)

    nchunk = bpw // CHUNK
    f = pl.kernel(
        functools.partial(_neumf_body, bpw, nc),
        out_type=jax.ShapeDtypeStruct((B,), jnp.float32),
        mesh=mesh,
        compiler_params=pltpu.CompilerParams(
            needs_layout_passes=False, use_tc_tiling_on_sc=False),
        scratch_types=[
            pltpu.VMEM((nchunk, CHUNK), jnp.int32),
            pltpu.VMEM((nchunk, CHUNK), jnp.int32),
            pltpu.VMEM((nchunk, CHUNK), jnp.int32),
            pltpu.VMEM((nchunk, CHUNK), jnp.int32),
            [pltpu.VMEM((CHUNK, 128), jnp.int32)] * 4,
            pltpu.VMEM((WFLAT,), jnp.float32),
            pltpu.VMEM((bpw,), jnp.float32),
            pltpu.SemaphoreType.DMA,
        ],
    )
    out = f(user2, item2, ugp, igp, ump, imp, wflat)
    return out.reshape(B, 1)


# speedup vs baseline: 32.7396x; 1.0881x over previous
import functools

import jax
import jax.numpy as jnp
from jax import lax
from jax.experimental import pallas as pl
from jax.experimental.pallas import tpu as pltpu
from jax.experimental.pallas import tpu_sc as plsc

B = 16384
EMB = 16
CHUNK = 128

V = 1000001
TBLK = 16384
GRP = TBLK // 8
LOG_TBLK = 14
LOG_GRP = 11
NSTEP = (V + TBLK - 1) // TBLK
PROWS = NSTEP * GRP

OFF_W1 = 0
OFF_W2 = 2048
OFF_W3 = 4096
OFF_W4 = 4608
OFF_WO = 4864
OFF_B1 = 4896
OFF_B2 = 4960
OFF_B3 = 4992
OFF_B4 = 5008
OFF_BO = 5024
WFLAT = 5056


def _detile_body(xug, xig, xum, xim, ou, oi):
    for g, m, o in ((xug, xum, ou), (xig, xim, oi)):
        gb = g[...]
        mb = m[...]
        lo = jnp.concatenate(
            [gb[:, p * GRP:(p + 1) * GRP] for p in range(8)], axis=0)
        hi = jnp.concatenate(
            [mb[:, p * GRP:(p + 1) * GRP] for p in range(8)], axis=0)
        au = lax.bitcast_convert_type(
            lo.astype(jnp.bfloat16), jnp.uint16).astype(jnp.uint32)
        bu = lax.bitcast_convert_type(
            hi.astype(jnp.bfloat16), jnp.uint16).astype(jnp.uint32)
        w = lax.bitcast_convert_type((bu << 16) | au, jnp.int32)
        o[...] = w.T


_detile = pl.pallas_call(
    _detile_body,
    grid=(NSTEP,),
    in_specs=[pl.BlockSpec((EMB, TBLK), lambda j: (0, j))] * 4,
    out_specs=[pl.BlockSpec((GRP, 128), lambda j: (j, 0))] * 2,
    out_shape=[jax.ShapeDtypeStruct((PROWS, 128), jnp.int32)] * 2,
)


def _neumf_body(bpw, nc, user_hbm, item_hbm, upk, ipk, wflat_hbm,
                out_hbm, uidx, iidx, uq, iq, bufs, w_v, out_v, sems):
    wid = lax.axis_index("s") * nc + lax.axis_index("c")
    nchunk = bpw // CHUNK
    base = wid * bpw
    iota = lax.iota(jnp.int32, 16)
    zeros = jnp.zeros((16,), jnp.float32)

    pltpu.sync_copy(user_hbm.at[pl.ds(wid * nchunk, nchunk)], uidx)
    pltpu.sync_copy(item_hbm.at[pl.ds(wid * nchunk, nchunk)], iidx)
    def packed_row(v):
        return (lax.shift_left(lax.shift_right_logical(v, LOG_TBLK), LOG_GRP)
                | (v & (GRP - 1)))

    for j in range(nchunk):
        for h in range(CHUNK // 16):
            sl = pl.ds(h * 16, 16)
            uq[j, sl] = packed_row(uidx[j, sl])
            iq[j, sl] = packed_row(iidx[j, sl])

    ubufs = bufs[0:2]
    ibufs = bufs[2:4]

    def fire(j, slot):
        return [
            pltpu.async_copy(upk.at[uq.at[j]], ubufs[slot], sems[slot]),
            pltpu.async_copy(ipk.at[iq.at[j]], ibufs[slot], sems[slot]),
        ]

    inflight = fire(0, 0)

    pltpu.sync_copy(wflat_hbm, w_v)

    wm = w_v[pl.ds(OFF_WO + 16, 16)]
    wg = w_v[pl.ds(OFF_WO, 16)]

    def matvec_half(w_off, ncols, h, s):
        acc = zeros
        for j in range(ncols):
            col = plsc.load_gather(w_v, [w_off + (h * 16 + iota) * ncols + j])
            acc = acc + s[j // 16][j % 16] * col
        return acc

    s4 = matvec_half(OFF_W4, 16, 0, [wm])
    s3 = [matvec_half(OFF_W3, 16, h, [s4]) for h in range(2)]
    s2 = [matvec_half(OFF_W2, 32, q, s3) for q in range(4)]
    s1 = [matvec_half(OFF_W1, 64, h, s2) for h in range(2)]
    vu, vi = s1[0], s1[1]

    cv = zeros
    for q in range(4):
        cv = cv + w_v[pl.ds(OFF_B1 + 16 * q, 16)] * s2[q]
    for h in range(2):
        cv = cv + w_v[pl.ds(OFF_B2 + 16 * h, 16)] * s3[h]
    cv = cv + w_v[pl.ds(OFF_B3, 16)] * s4
    cv = cv + w_v[pl.ds(OFF_B4, 16)] * wm
    c = jnp.sum(cv) + w_v[pl.ds(OFF_BO, 16)][0]

    for j in range(nchunk):
        for cp in inflight:
            cp.wait()
        if j + 1 < nchunk:
            nxt = fire(j + 1, (j + 1) % 2)
        ubuf = ubufs[j % 2]
        ibuf = ibufs[j % 2]

        def blk_step(b, carry, j=j, ubuf=ubuf, ibuf=ibuf):
            sl = pl.ds(b * 16, 16)
            rows = b * 16 + iota
            ru = (lax.shift_right_logical(uidx[j, sl], LOG_GRP) & 7) * 16
            ri = (lax.shift_right_logical(iidx[j, sl], LOG_GRP) & 7) * 16
            acc = jnp.full((16,), c, jnp.float32)
            hi_mask = jnp.full((16,), -65536, jnp.int32)

            for k in range(EMB):
                uw = plsc.load_gather(ubuf, [rows, ru + k])
                iw = plsc.load_gather(ibuf, [rows, ri + k])
                ug_ = plsc.bitcast(lax.shift_left(uw, 16), jnp.float32)
                um_ = plsc.bitcast(uw & hi_mask, jnp.float32)
                ig_ = plsc.bitcast(lax.shift_left(iw, 16), jnp.float32)
                im_ = plsc.bitcast(iw & hi_mask, jnp.float32)
                acc = acc + (ug_ * ig_) * wg[k] + um_ * vu[k] + im_ * vi[k]
            out_v[pl.ds(j * CHUNK + b * 16, 16)] = 1.0 / (1.0 + jnp.exp(-acc))
            return carry

        lax.fori_loop(0, CHUNK // 16, blk_step, 0)
        if j + 1 < nchunk:
            inflight = nxt

    pltpu.sync_copy(out_v, out_hbm.at[pl.ds(base, bpw)])


def kernel(user, item, user_GMF, item_GMF, user_MLP, item_MLP,
           W1, b1, W2, b2, W3, b3, W4, b4, Wo, bo):
    mesh = plsc.VectorSubcoreMesh(core_axis_name="c", subcore_axis_name="s")
    nc, ns = mesh.num_cores, mesh.num_subcores
    nw = nc * ns
    assert B % (nw * CHUNK) == 0
    bpw = B // nw

    pad = jnp.zeros((WFLAT - 5025,), jnp.float32)
    wflat = jnp.concatenate([
        W1.reshape(-1), W2.reshape(-1), W3.reshape(-1), W4.reshape(-1),
        Wo.reshape(-1), b1, b2, b3, b4, bo, pad])
    user2 = user.astype(jnp.int32).reshape(B // CHUNK, CHUNK)
    item2 = item.astype(jnp.int32).reshape(B // CHUNK, CHUNK)

    upk, ipk = _detile(user_GMF.T, item_GMF.T, user_MLP.T, item_MLP.T)

    nchunk = bpw // CHUNK
    f = pl.kernel(
        functools.partial(_neumf_body, bpw, nc),
        out_type=jax.ShapeDtypeStruct((B,), jnp.float32),
        mesh=mesh,
        compiler_params=pltpu.CompilerParams(
            needs_layout_passes=False, use_tc_tiling_on_sc=False),
        scratch_types=[
            pltpu.VMEM((nchunk, CHUNK), jnp.int32),
            pltpu.VMEM((nchunk, CHUNK), jnp.int32),
            pltpu.VMEM((nchunk, CHUNK), jnp.int32),
            pltpu.VMEM((nchunk, CHUNK), jnp.int32),
            [pltpu.VMEM((CHUNK, 128), jnp.int32)] * 4,
            pltpu.VMEM((WFLAT,), jnp.float32),
            pltpu.VMEM((bpw,), jnp.float32),
            [pltpu.SemaphoreType.DMA] * 2,
        ],
    )
    out = f(user2, item2, upk, ipk, wflat)
    return out.reshape(B, 1)


# speedup vs baseline: 34.2384x; 1.0458x over previous
import functools

import jax
import jax.numpy as jnp
from jax import lax
from jax.experimental import pallas as pl
from jax.experimental.pallas import tpu as pltpu
from jax.experimental.pallas import tpu_sc as plsc

B = 16384
EMB = 16
CHUNK = 128

V = 1000001
TBLK = 32768
GRP = TBLK // 8
LOG_TBLK = 15
LOG_GRP = 12
NSTEP = (V + TBLK - 1) // TBLK
PROWS = NSTEP * GRP

OFF_W1 = 0
OFF_W2 = 2048
OFF_W3 = 4096
OFF_W4 = 4608
OFF_WO = 4864
OFF_B1 = 4896
OFF_B2 = 4960
OFF_B3 = 4992
OFF_B4 = 5008
OFF_BO = 5024
WFLAT = 5056


def _detile_body(xug, xig, xum, xim, ou, oi):
    for g, m, o in ((xug, xum, ou), (xig, xim, oi)):
        gb = g[...]
        mb = m[...]
        lo = jnp.concatenate(
            [gb[:, p * GRP:(p + 1) * GRP] for p in range(8)], axis=0)
        hi = jnp.concatenate(
            [mb[:, p * GRP:(p + 1) * GRP] for p in range(8)], axis=0)
        au = lax.bitcast_convert_type(
            lo.astype(jnp.bfloat16), jnp.uint16).astype(jnp.uint32)
        bu = lax.bitcast_convert_type(
            hi.astype(jnp.bfloat16), jnp.uint16).astype(jnp.uint32)
        w = lax.bitcast_convert_type((bu << 16) | au, jnp.int32)
        o[...] = w.T


_detile = pl.pallas_call(
    _detile_body,
    grid=(NSTEP,),
    in_specs=[pl.BlockSpec((EMB, TBLK), lambda j: (0, j))] * 4,
    out_specs=[pl.BlockSpec((GRP, 128), lambda j: (j, 0))] * 2,
    out_shape=[jax.ShapeDtypeStruct((PROWS, 128), jnp.int32)] * 2,
)


def _neumf_body(bpw, nc, user_hbm, item_hbm, upk, ipk, wflat_hbm,
                out_hbm, uidx, iidx, uq, iq, bufs, w_v, out_v, sems):
    wid = lax.axis_index("s") * nc + lax.axis_index("c")
    nchunk = bpw // CHUNK
    base = wid * bpw
    iota = lax.iota(jnp.int32, 16)
    zeros = jnp.zeros((16,), jnp.float32)

    pltpu.sync_copy(user_hbm.at[pl.ds(wid * nchunk, nchunk)], uidx)
    pltpu.sync_copy(item_hbm.at[pl.ds(wid * nchunk, nchunk)], iidx)
    def packed_row(v):
        return (lax.shift_left(lax.shift_right_logical(v, LOG_TBLK), LOG_GRP)
                | (v & (GRP - 1)))

    for j in range(nchunk):
        for h in range(CHUNK // 16):
            sl = pl.ds(h * 16, 16)
            uq[j, sl] = packed_row(uidx[j, sl])
            iq[j, sl] = packed_row(iidx[j, sl])

    ubufs = bufs[0:2]
    ibufs = bufs[2:4]

    def fire(j, slot):
        return [
            pltpu.async_copy(upk.at[uq.at[j]], ubufs[slot], sems[slot]),
            pltpu.async_copy(ipk.at[iq.at[j]], ibufs[slot], sems[slot]),
        ]

    inflight = fire(0, 0)

    pltpu.sync_copy(wflat_hbm, w_v)

    wm = w_v[pl.ds(OFF_WO + 16, 16)]
    wg = w_v[pl.ds(OFF_WO, 16)]

    def matvec_half(w_off, ncols, h, s):
        acc = zeros
        for j in range(ncols):
            col = plsc.load_gather(w_v, [w_off + (h * 16 + iota) * ncols + j])
            acc = acc + s[j // 16][j % 16] * col
        return acc

    s4 = matvec_half(OFF_W4, 16, 0, [wm])
    s3 = [matvec_half(OFF_W3, 16, h, [s4]) for h in range(2)]
    s2 = [matvec_half(OFF_W2, 32, q, s3) for q in range(4)]
    s1 = [matvec_half(OFF_W1, 64, h, s2) for h in range(2)]
    vu, vi = s1[0], s1[1]

    cv = zeros
    for q in range(4):
        cv = cv + w_v[pl.ds(OFF_B1 + 16 * q, 16)] * s2[q]
    for h in range(2):
        cv = cv + w_v[pl.ds(OFF_B2 + 16 * h, 16)] * s3[h]
    cv = cv + w_v[pl.ds(OFF_B3, 16)] * s4
    cv = cv + w_v[pl.ds(OFF_B4, 16)] * wm
    c = jnp.sum(cv) + w_v[pl.ds(OFF_BO, 16)][0]

    for j in range(nchunk):
        for cp in inflight:
            cp.wait()
        if j + 1 < nchunk:
            nxt = fire(j + 1, (j + 1) % 2)
        ubuf = ubufs[j % 2]
        ibuf = ibufs[j % 2]

        def blk_step(b, carry, j=j, ubuf=ubuf, ibuf=ibuf):
            sl = pl.ds(b * 16, 16)
            rows = b * 16 + iota
            ru = (lax.shift_right_logical(uidx[j, sl], LOG_GRP) & 7) * 16
            ri = (lax.shift_right_logical(iidx[j, sl], LOG_GRP) & 7) * 16
            acc = jnp.full((16,), c, jnp.float32)
            hi_mask = jnp.full((16,), -65536, jnp.int32)

            for k in range(EMB):
                uw = plsc.load_gather(ubuf, [rows, ru + k])
                iw = plsc.load_gather(ibuf, [rows, ri + k])
                ug_ = plsc.bitcast(lax.shift_left(uw, 16), jnp.float32)
                um_ = plsc.bitcast(uw & hi_mask, jnp.float32)
                ig_ = plsc.bitcast(lax.shift_left(iw, 16), jnp.float32)
                im_ = plsc.bitcast(iw & hi_mask, jnp.float32)
                acc = acc + (ug_ * ig_) * wg[k] + um_ * vu[k] + im_ * vi[k]
            out_v[pl.ds(j * CHUNK + b * 16, 16)] = 1.0 / (1.0 + jnp.exp(-acc))
            return carry

        lax.fori_loop(0, CHUNK // 16, blk_step, 0)
        if j + 1 < nchunk:
            inflight = nxt

    pltpu.sync_copy(out_v, out_hbm.at[pl.ds(base, bpw)])


def kernel(user, item, user_GMF, item_GMF, user_MLP, item_MLP,
           W1, b1, W2, b2, W3, b3, W4, b4, Wo, bo):
    mesh = plsc.VectorSubcoreMesh(core_axis_name="c", subcore_axis_name="s")
    nc, ns = mesh.num_cores, mesh.num_subcores
    nw = nc * ns
    assert B % (nw * CHUNK) == 0
    bpw = B // nw

    pad = jnp.zeros((WFLAT - 5025,), jnp.float32)
    wflat = jnp.concatenate([
        W1.reshape(-1), W2.reshape(-1), W3.reshape(-1), W4.reshape(-1),
        Wo.reshape(-1), b1, b2, b3, b4, bo, pad])
    user2 = user.astype(jnp.int32).reshape(B // CHUNK, CHUNK)
    item2 = item.astype(jnp.int32).reshape(B // CHUNK, CHUNK)

    upk, ipk = _detile(user_GMF.T, item_GMF.T, user_MLP.T, item_MLP.T)

    nchunk = bpw // CHUNK
    f = pl.kernel(
        functools.partial(_neumf_body, bpw, nc),
        out_type=jax.ShapeDtypeStruct((B,), jnp.float32),
        mesh=mesh,
        compiler_params=pltpu.CompilerParams(
            needs_layout_passes=False, use_tc_tiling_on_sc=False),
        scratch_types=[
            pltpu.VMEM((nchunk, CHUNK), jnp.int32),
            pltpu.VMEM((nchunk, CHUNK), jnp.int32),
            pltpu.VMEM((nchunk, CHUNK), jnp.int32),
            pltpu.VMEM((nchunk, CHUNK), jnp.int32),
            [pltpu.VMEM((CHUNK, 128), jnp.int32)] * 4,
            pltpu.VMEM((WFLAT,), jnp.float32),
            pltpu.VMEM((bpw,), jnp.float32),
            [pltpu.SemaphoreType.DMA] * 2,
        ],
    )
    out = f(user2, item2, upk, ipk, wflat)
    return out.reshape(B, 1)
